# baseline (speedup 1.0000x reference)
import functools

import jax
import jax.numpy as jnp
from jax import lax
from jax.experimental import pallas as pl
from jax.experimental.pallas import tpu as pltpu
from jax.experimental.pallas import tpu_sc as plsc

B, L, D = 16384, 200, 64
N = B * L
SCALE = 8.0

NC, NS = 2, 16
NW = NC * NS
B_PER_W = B // NW

HL = L // 2
BBLK = 4
GROUPS = B_PER_W // BBLK
NBUF = 2

_PRE_ROWS = 1000


def _prescale_body(table_ref, out_ref):
    i = pl.program_id(0)
    row = lax.broadcasted_iota(jnp.int32, table_ref.shape, 0) + i * _PRE_ROWS
    out_ref[...] = jnp.where(row == 0, 0.0, table_ref[...] * SCALE)


def _prescale(table):
    v, d = table.shape
    return pl.pallas_call(
        _prescale_body,
        grid=(v // _PRE_ROWS,),
        in_specs=[pl.BlockSpec((_PRE_ROWS, d), lambda i: (i, 0))],
        out_specs=pl.BlockSpec((_PRE_ROWS, d), lambda i: (i, 0)),
        out_shape=jax.ShapeDtypeStruct((v, d), jnp.float32),
    )(table)


_mesh = plsc.VectorSubcoreMesh(core_axis_name="c", subcore_axis_name="s")


@functools.partial(
    pl.kernel,
    mesh=_mesh,
    out_type=jax.ShapeDtypeStruct((B, L, D), jnp.float32),
    scratch_types=[
        [pltpu.VMEM((2 * BBLK, HL), jnp.int32) for _ in range(NBUF)],
        [pltpu.VMEM((BBLK, L, D), jnp.float32) for _ in range(NBUF)],
        [pltpu.SemaphoreType.DMA for _ in range(NBUF)],
        [pltpu.SemaphoreType.DMA for _ in range(NBUF)],
    ],
    compiler_params=pltpu.CompilerParams(use_tc_tiling_on_sc=False),
)
def _gather(table_hbm, idx_hbm, out_hbm, idx_bufs, row_bufs, gsems, ssems):
    wid = lax.axis_index("s") * NC + lax.axis_index("c")
    b_base = wid * B_PER_W

    def fire_gather(g, b):
        pltpu.sync_copy(
            idx_hbm.at[pl.ds(2 * (b_base + g * BBLK), 2 * BBLK)], idx_bufs[b]
        )
        for k in range(BBLK):
            for h in range(2):
                pltpu.async_copy(
                    table_hbm.at[idx_bufs[b].at[2 * k + h]],
                    row_bufs[b].at[k, pl.ds(h * HL, HL), :],
                    gsems[b],
                )

    def wait_gather(b):
        for k in range(BBLK):
            for h in range(2):
                pltpu.make_async_copy(
                    table_hbm.at[idx_bufs[b].at[2 * k + h]],
                    row_bufs[b].at[k, pl.ds(h * HL, HL), :],
                    gsems[b],
                ).wait()

    for b in range(NBUF):
        fire_gather(b, b)

    def body(t, carry):
        for b in range(NBUF):
            g = t * NBUF + b
            wait_gather(b)
            st = pltpu.async_copy(
                row_bufs[b],
                out_hbm.at[pl.ds(b_base + g * BBLK, BBLK)],
                ssems[b],
            )
            st.wait()

            @pl.when(g + NBUF < GROUPS)
            def _():
                fire_gather(g + NBUF, b)

        return carry

    lax.fori_loop(0, GROUPS // NBUF, body, 0)


def kernel(inputs, table):
    scaled = _prescale(table.astype(jnp.float32))
    idx2d = inputs.reshape(2 * B, HL).astype(jnp.int32)
    return _gather(scaled, idx2d)
